# baseline (speedup 1.0000x reference)
import functools

import jax
import jax.numpy as jnp
from jax.experimental import pallas as pl

NPAD = 10240
BM = 1024
BK = 2048


def _split(s):
    hi = s.astype(jnp.bfloat16)
    lo = (s - hi.astype(jnp.float32)).astype(jnp.bfloat16)
    return hi, lo


def _prep_body(v_ref, si_ref, hl_ref):
    hi, lo = _split(v_ref[...] * si_ref[...])
    hl_ref[...] = jnp.concatenate([hi, lo], axis=1)


def _prep(v, si):
    w = v.shape[1]
    return pl.pallas_call(
        _prep_body,
        grid=(NPAD // BM,),
        in_specs=[
            pl.BlockSpec((BM, w), lambda i: (i, 0)),
            pl.BlockSpec((BM, 1), lambda i: (i, 0)),
        ],
        out_specs=pl.BlockSpec((BM, 2 * w), lambda i: (i, 0)),
        out_shape=jax.ShapeDtypeStruct((NPAD, 2 * w), jnp.bfloat16),
    )(v, si.reshape(NPAD, 1))


def _spmv_body(*refs, mode, emit, convert):
    k = pl.program_id(1)
    nk = pl.num_programs(1)
    if mode == "gcn":
        (so_ref, a_ref, hl_ref, sl_ref, vown_ref, b_ref), rest = \
            refs[:6], refs[6:]
    elif mode == "cheb":
        (so_ref, a_ref, hl_ref, aux_ref), rest = refs[:4], refs[4:]
    else:
        (so_ref, a_ref, hl_ref), rest = refs[:3], refs[3:]
    if convert:
        abf_ref = rest[-1]
        rest = rest[:-1]
    if emit:
        sin_ref = rest[0]
        out_ref, ohl_ref = rest[1:]
    else:
        (out_ref,) = rest

    w = hl_ref.shape[1] // 2
    a = a_ref[...]
    if convert:
        a = a.astype(jnp.bfloat16)
        abf_ref[...] = a
    both = jnp.dot(a, hl_ref[...], preferred_element_type=jnp.float32)
    part = both[:, :w] + both[:, w:]

    @pl.when(k == 0)
    def _():
        out_ref[...] = part

    @pl.when(k > 0)
    def _():
        out_ref[...] += part

    @pl.when(k == nk - 1)
    def _():
        base = out_ref[...] * so_ref[...]
        if mode == "scale":
            res = base
        elif mode == "cheb":
            res = 2.0 * base - aux_ref[...]
        else:
            res = jax.nn.relu(base + sl_ref[...] * vown_ref[...] + b_ref[...])
        out_ref[...] = res
        if emit:
            hi, lo = _split(res * sin_ref[...])
            ohl_ref[...] = jnp.concatenate([hi, lo], axis=1)


def _spmv(a, hl, so, mode, aux=None, sl=None, vown=None, b=None,
          si_next=None, convert=False):
    w = hl.shape[1] // 2
    grid = (NPAD // BM, NPAD // BK)
    emit = si_next is not None
    in_specs = [
        pl.BlockSpec((BM, 1), lambda i, k: (i, 0)),
        pl.BlockSpec((BM, BK), lambda i, k: (i, k)),
        pl.BlockSpec((BK, 2 * w), lambda i, k: (k, 0)),
    ]
    args = [so.reshape(NPAD, 1), a, hl]
    if mode == "cheb":
        in_specs.append(pl.BlockSpec((BM, w), lambda i, k: (i, 0)))
        args.append(aux)
    elif mode == "gcn":
        in_specs += [
            pl.BlockSpec((BM, 1), lambda i, k: (i, 0)),
            pl.BlockSpec((BM, w), lambda i, k: (i, 0)),
            pl.BlockSpec((1, w), lambda i, k: (0, 0)),
        ]
        args += [sl.reshape(NPAD, 1), vown, b.reshape(1, w)]
    if emit:
        in_specs.append(pl.BlockSpec((BM, 1), lambda i, k: (i, 0)))
        args.append(si_next.reshape(NPAD, 1))
        out_specs = [pl.BlockSpec((BM, w), lambda i, k: (i, 0)),
                     pl.BlockSpec((BM, 2 * w), lambda i, k: (i, 0))]
        out_shape = [jax.ShapeDtypeStruct((NPAD, w), jnp.float32),
                     jax.ShapeDtypeStruct((NPAD, 2 * w), jnp.bfloat16)]
    else:
        out_specs = [pl.BlockSpec((BM, w), lambda i, k: (i, 0))]
        out_shape = [jax.ShapeDtypeStruct((NPAD, w), jnp.float32)]
    if convert:
        out_specs = list(out_specs) if isinstance(out_specs, list) else [out_specs]
        out_specs.append(pl.BlockSpec((BM, BK), lambda i, k: (i, k)))
        out_shape = list(out_shape) if isinstance(out_shape, list) else [out_shape]
        out_shape.append(jax.ShapeDtypeStruct((NPAD, NPAD), jnp.bfloat16))

    res = pl.pallas_call(
        functools.partial(_spmv_body, mode=mode, emit=emit, convert=convert),
        grid=grid,
        in_specs=in_specs,
        out_specs=out_specs,
        out_shape=out_shape,
    )(*args)
    if not emit and not convert:
        return res[0]
    return res


def _mm_body(*refs, relu, emit):
    if emit:
        x_ref, w_ref, b_ref, si_ref, out_ref, ohl_ref = refs
    else:
        x_ref, w_ref, b_ref, out_ref = refs
    r = jnp.dot(x_ref[...], w_ref[...],
                preferred_element_type=jnp.float32) + b_ref[...]
    if relu:
        r = jax.nn.relu(r)
    out_ref[...] = r
    if emit:
        hi, lo = _split(r * si_ref[...])
        ohl_ref[...] = jnp.concatenate([hi, lo], axis=1)


def _mm(x, w, b, relu, si_next=None):
    fin, fout = w.shape
    emit = si_next is not None
    in_specs = [
        pl.BlockSpec((BM, fin), lambda i: (i, 0)),
        pl.BlockSpec((fin, fout), lambda i: (0, 0)),
        pl.BlockSpec((1, fout), lambda i: (0, 0)),
    ]
    args = [x, w, b.reshape(1, fout)]
    if emit:
        in_specs.append(pl.BlockSpec((BM, 1), lambda i: (i, 0)))
        args.append(si_next.reshape(NPAD, 1))
        out_specs = [pl.BlockSpec((BM, fout), lambda i: (i, 0)),
                     pl.BlockSpec((BM, 2 * fout), lambda i: (i, 0))]
        out_shape = [jax.ShapeDtypeStruct((NPAD, fout), jnp.float32),
                     jax.ShapeDtypeStruct((NPAD, 2 * fout), jnp.bfloat16)]
    else:
        out_specs = pl.BlockSpec((BM, fout), lambda i: (i, 0))
        out_shape = jax.ShapeDtypeStruct((NPAD, fout), jnp.float32)
    return pl.pallas_call(
        functools.partial(_mm_body, relu=relu, emit=emit),
        grid=(NPAD // BM,),
        in_specs=in_specs,
        out_specs=out_specs,
        out_shape=out_shape,
    )(*args)


def _pool_body(h_ref, batch_ref, gate_w_ref, gate_b_ref, fc_w_ref, fc_b_ref,
               out_ref, *, nb):
    h = h_ref[...]
    g = jnp.dot(h, gate_w_ref[...],
                preferred_element_type=jnp.float32) + gate_b_ref[...]
    seg = jax.lax.broadcasted_iota(jnp.int32, (h.shape[0], nb), 1)
    m = batch_ref[...] == seg
    neg = jnp.float32(-jnp.inf)
    gmax = jnp.max(jnp.where(m, g, neg), axis=0, keepdims=True)
    gmax = jnp.where(jnp.isfinite(gmax), gmax, 0.0)
    ge = jnp.where(m, jnp.exp(g - gmax), 0.0)
    gs = jnp.sum(ge, axis=0, keepdims=True)
    att = ge / jnp.maximum(gs, 1e-12)
    pooled = jax.lax.dot_general(att, h, (((0,), (0,)), ((), ())),
                                 preferred_element_type=jnp.float32)
    logits = jnp.dot(pooled, fc_w_ref[...],
                     preferred_element_type=jnp.float32) + fc_b_ref[...]
    mx = jnp.max(logits, axis=1, keepdims=True)
    lse = mx + jnp.log(jnp.sum(jnp.exp(logits - mx), axis=1, keepdims=True))
    out_ref[...] = logits - lse


def _pool(h, batch_padded, gate_w, gate_b, fc_w, fc_b, nb):
    c = fc_w.shape[1]
    return pl.pallas_call(
        functools.partial(_pool_body, nb=nb),
        out_shape=jax.ShapeDtypeStruct((nb, c), jnp.float32),
    )(h, batch_padded.reshape(NPAD, 1), gate_w, gate_b.reshape(1, 1),
      fc_w, fc_b.reshape(1, c))


def kernel(x, edge_index, batch, cheb_W, cheb_b, gcn1_W, gcn1_b,
           gcn2_W, gcn2_b, gate_w, gate_b, fc_W, fc_b):
    n, f = x.shape
    nb = 8
    row, col = edge_index[0], edge_index[1]

    base = NPAD * NPAD
    flat = col.astype(jnp.int32) * NPAD + row.astype(jnp.int32)
    idx = jnp.concatenate([flat, base + row, base + NPAD + col])
    buf = (jnp.zeros(((NPAD + 8) * NPAD,), jnp.float32)
           .at[idx].add(1.0).reshape(NPAD + 8, NPAD))
    deg_r = buf[NPAD]
    deg_c = buf[NPAD + 1]
    dinv = jnp.where(deg_r > 0, jax.lax.rsqrt(jnp.where(deg_r > 0, deg_r, 1.0)), 0.0)
    dgc = jax.lax.rsqrt(deg_c + 1.0)
    dgc2 = dgc * dgc

    xp = jnp.zeros((NPAD, f), jnp.float32).at[:n].set(x)
    batch_p = jnp.full((NPAD,), nb, jnp.int32).at[:n].set(batch)

    tx0 = xp
    hl0 = _prep(tx0, dinv)
    tx1, hl1, a = _spmv(buf, hl0, so=-dinv, mode="scale", si_next=dinv,
                        convert=True)
    tx2, hl2 = _spmv(a, hl1, so=-dinv, mode="cheb", aux=tx0, si_next=dinv)
    tx3, hl3 = _spmv(a, hl2, so=-dinv, mode="cheb", aux=tx1, si_next=dinv)
    tx4 = _spmv(a, hl3, so=-dinv, mode="cheb", aux=tx2)
    cat = jnp.concatenate([tx0, tx1, tx2, tx3, tx4], axis=1)
    wcat = cheb_W.reshape(5 * f, cheb_W.shape[2])
    h1 = _mm(cat, wcat, cheb_b, relu=True)

    z1 = jnp.zeros((gcn1_W.shape[1],), jnp.float32)
    z2 = jnp.zeros((gcn2_W.shape[1],), jnp.float32)
    vw1, vhl1 = _mm(h1, gcn1_W, z1, relu=False, si_next=dgc)
    h2 = _spmv(a, vhl1, so=dgc, mode="gcn", sl=dgc2, vown=vw1, b=gcn1_b)
    vw2, vhl2 = _mm(h2, gcn2_W, z2, relu=False, si_next=dgc)
    h3 = _spmv(a, vhl2, so=dgc, mode="gcn", sl=dgc2, vown=vw2, b=gcn2_b)

    return _pool(h3, batch_p, gate_w, gate_b, fc_W, fc_b, nb)


# speedup vs baseline: 1.0551x; 1.0551x over previous
import functools

import jax
import jax.numpy as jnp
from jax.experimental import pallas as pl

NPAD = 10240
BM = 1024
BK = 2048
SPLIT2 = False


def _split(s):
    hi = s.astype(jnp.bfloat16)
    lo = (s - hi.astype(jnp.float32)).astype(jnp.bfloat16)
    return hi, lo


def _emit_hl(s):
    if SPLIT2:
        hi, lo = _split(s)
        return jnp.concatenate([hi, lo], axis=1)
    return s.astype(jnp.bfloat16)


def _prep_body(v_ref, si_ref, hl_ref):
    hl_ref[...] = _emit_hl(v_ref[...] * si_ref[...])


def _prep(v, si):
    w = v.shape[1]
    return pl.pallas_call(
        _prep_body,
        grid=(NPAD // BM,),
        in_specs=[
            pl.BlockSpec((BM, w), lambda i: (i, 0)),
            pl.BlockSpec((BM, 1), lambda i: (i, 0)),
        ],
        out_specs=pl.BlockSpec((BM, (2 if SPLIT2 else 1) * w), lambda i: (i, 0)),
        out_shape=jax.ShapeDtypeStruct((NPAD, (2 if SPLIT2 else 1) * w), jnp.bfloat16),
    )(v, si.reshape(NPAD, 1))


def _spmv_body(*refs, mode, emit, convert):
    k = pl.program_id(1)
    nk = pl.num_programs(1)
    if mode == "gcn":
        (so_ref, a_ref, hl_ref, sl_ref, vown_ref, b_ref), rest = \
            refs[:6], refs[6:]
    elif mode == "cheb":
        (so_ref, a_ref, hl_ref, aux_ref), rest = refs[:4], refs[4:]
    else:
        (so_ref, a_ref, hl_ref), rest = refs[:3], refs[3:]
    if convert:
        abf_ref = rest[-1]
        rest = rest[:-1]
    if emit:
        sin_ref = rest[0]
        out_ref, ohl_ref = rest[1:]
    else:
        (out_ref,) = rest

    a = a_ref[...]
    if convert:
        a = a.astype(jnp.bfloat16)
        abf_ref[...] = a
    both = jnp.dot(a, hl_ref[...], preferred_element_type=jnp.float32)
    if SPLIT2:
        w = hl_ref.shape[1] // 2
        part = both[:, :w] + both[:, w:]
    else:
        part = both

    @pl.when(k == 0)
    def _():
        out_ref[...] = part

    @pl.when(k > 0)
    def _():
        out_ref[...] += part

    @pl.when(k == nk - 1)
    def _():
        base = out_ref[...] * so_ref[...]
        if mode == "scale":
            res = base
        elif mode == "cheb":
            res = 2.0 * base - aux_ref[...]
        else:
            res = jax.nn.relu(base + sl_ref[...] * vown_ref[...] + b_ref[...])
        out_ref[...] = res
        if emit:
            ohl_ref[...] = _emit_hl(res * sin_ref[...])


def _spmv(a, hl, so, mode, aux=None, sl=None, vown=None, b=None,
          si_next=None, convert=False):
    w = hl.shape[1] // (2 if SPLIT2 else 1)
    grid = (NPAD // BM, NPAD // BK)
    emit = si_next is not None
    in_specs = [
        pl.BlockSpec((BM, 1), lambda i, k: (i, 0)),
        pl.BlockSpec((BM, BK), lambda i, k: (i, k)),
        pl.BlockSpec((BK, (2 if SPLIT2 else 1) * w), lambda i, k: (k, 0)),
    ]
    args = [so.reshape(NPAD, 1), a, hl]
    if mode == "cheb":
        in_specs.append(pl.BlockSpec((BM, w), lambda i, k: (i, 0)))
        args.append(aux)
    elif mode == "gcn":
        in_specs += [
            pl.BlockSpec((BM, 1), lambda i, k: (i, 0)),
            pl.BlockSpec((BM, w), lambda i, k: (i, 0)),
            pl.BlockSpec((1, w), lambda i, k: (0, 0)),
        ]
        args += [sl.reshape(NPAD, 1), vown, b.reshape(1, w)]
    if emit:
        in_specs.append(pl.BlockSpec((BM, 1), lambda i, k: (i, 0)))
        args.append(si_next.reshape(NPAD, 1))
        out_specs = [pl.BlockSpec((BM, w), lambda i, k: (i, 0)),
                     pl.BlockSpec((BM, (2 if SPLIT2 else 1) * w),
                                  lambda i, k: (i, 0))]
        out_shape = [jax.ShapeDtypeStruct((NPAD, w), jnp.float32),
                     jax.ShapeDtypeStruct((NPAD, (2 if SPLIT2 else 1) * w),
                                          jnp.bfloat16)]
    else:
        out_specs = [pl.BlockSpec((BM, w), lambda i, k: (i, 0))]
        out_shape = [jax.ShapeDtypeStruct((NPAD, w), jnp.float32)]
    if convert:
        out_specs = list(out_specs) if isinstance(out_specs, list) else [out_specs]
        out_specs.append(pl.BlockSpec((BM, BK), lambda i, k: (i, k)))
        out_shape = list(out_shape) if isinstance(out_shape, list) else [out_shape]
        out_shape.append(jax.ShapeDtypeStruct((NPAD, NPAD), jnp.bfloat16))

    res = pl.pallas_call(
        functools.partial(_spmv_body, mode=mode, emit=emit, convert=convert),
        grid=grid,
        in_specs=in_specs,
        out_specs=out_specs,
        out_shape=out_shape,
    )(*args)
    if not emit and not convert:
        return res[0]
    return res


def _mm_body(*refs, relu, emit):
    if emit:
        x_ref, w_ref, b_ref, si_ref, out_ref, ohl_ref = refs
    else:
        x_ref, w_ref, b_ref, out_ref = refs
    r = jnp.dot(x_ref[...], w_ref[...],
                preferred_element_type=jnp.float32) + b_ref[...]
    if relu:
        r = jax.nn.relu(r)
    out_ref[...] = r
    if emit:
        ohl_ref[...] = _emit_hl(r * si_ref[...])


def _mm(x, w, b, relu, si_next=None):
    fin, fout = w.shape
    emit = si_next is not None
    in_specs = [
        pl.BlockSpec((BM, fin), lambda i: (i, 0)),
        pl.BlockSpec((fin, fout), lambda i: (0, 0)),
        pl.BlockSpec((1, fout), lambda i: (0, 0)),
    ]
    args = [x, w, b.reshape(1, fout)]
    if emit:
        in_specs.append(pl.BlockSpec((BM, 1), lambda i: (i, 0)))
        args.append(si_next.reshape(NPAD, 1))
        out_specs = [pl.BlockSpec((BM, fout), lambda i: (i, 0)),
                     pl.BlockSpec((BM, (2 if SPLIT2 else 1) * fout),
                                  lambda i: (i, 0))]
        out_shape = [jax.ShapeDtypeStruct((NPAD, fout), jnp.float32),
                     jax.ShapeDtypeStruct((NPAD, (2 if SPLIT2 else 1) * fout),
                                          jnp.bfloat16)]
    else:
        out_specs = pl.BlockSpec((BM, fout), lambda i: (i, 0))
        out_shape = jax.ShapeDtypeStruct((NPAD, fout), jnp.float32)
    return pl.pallas_call(
        functools.partial(_mm_body, relu=relu, emit=emit),
        grid=(NPAD // BM,),
        in_specs=in_specs,
        out_specs=out_specs,
        out_shape=out_shape,
    )(*args)


def _pool_body(h_ref, batch_ref, gate_w_ref, gate_b_ref, fc_w_ref, fc_b_ref,
               out_ref, *, nb):
    h = h_ref[...]
    g = jnp.dot(h, gate_w_ref[...],
                preferred_element_type=jnp.float32) + gate_b_ref[...]
    seg = jax.lax.broadcasted_iota(jnp.int32, (h.shape[0], nb), 1)
    m = batch_ref[...] == seg
    neg = jnp.float32(-jnp.inf)
    gmax = jnp.max(jnp.where(m, g, neg), axis=0, keepdims=True)
    gmax = jnp.where(jnp.isfinite(gmax), gmax, 0.0)
    ge = jnp.where(m, jnp.exp(g - gmax), 0.0)
    gs = jnp.sum(ge, axis=0, keepdims=True)
    att = ge / jnp.maximum(gs, 1e-12)
    pooled = jax.lax.dot_general(att, h, (((0,), (0,)), ((), ())),
                                 preferred_element_type=jnp.float32)
    logits = jnp.dot(pooled, fc_w_ref[...],
                     preferred_element_type=jnp.float32) + fc_b_ref[...]
    mx = jnp.max(logits, axis=1, keepdims=True)
    lse = mx + jnp.log(jnp.sum(jnp.exp(logits - mx), axis=1, keepdims=True))
    out_ref[...] = logits - lse


def _pool(h, batch_padded, gate_w, gate_b, fc_w, fc_b, nb):
    c = fc_w.shape[1]
    return pl.pallas_call(
        functools.partial(_pool_body, nb=nb),
        out_shape=jax.ShapeDtypeStruct((nb, c), jnp.float32),
    )(h, batch_padded.reshape(NPAD, 1), gate_w, gate_b.reshape(1, 1),
      fc_w, fc_b.reshape(1, c))


def kernel(x, edge_index, batch, cheb_W, cheb_b, gcn1_W, gcn1_b,
           gcn2_W, gcn2_b, gate_w, gate_b, fc_W, fc_b):
    n, f = x.shape
    nb = 8
    row, col = edge_index[0], edge_index[1]

    base = NPAD * NPAD
    flat = col.astype(jnp.int32) * NPAD + row.astype(jnp.int32)
    idx = jnp.concatenate([flat, base + row, base + NPAD + col])
    buf = (jnp.zeros(((NPAD + 8) * NPAD,), jnp.float32)
           .at[idx].add(1.0).reshape(NPAD + 8, NPAD))
    deg_r = buf[NPAD]
    deg_c = buf[NPAD + 1]
    dinv = jnp.where(deg_r > 0, jax.lax.rsqrt(jnp.where(deg_r > 0, deg_r, 1.0)), 0.0)
    dgc = jax.lax.rsqrt(deg_c + 1.0)
    dgc2 = dgc * dgc

    xp = jnp.zeros((NPAD, f), jnp.float32).at[:n].set(x)
    batch_p = jnp.full((NPAD,), nb, jnp.int32).at[:n].set(batch)

    tx0 = xp
    hl0 = _prep(tx0, dinv)
    tx1, hl1, a = _spmv(buf, hl0, so=-dinv, mode="scale", si_next=dinv,
                        convert=True)
    tx2, hl2 = _spmv(a, hl1, so=-dinv, mode="cheb", aux=tx0, si_next=dinv)
    tx3, hl3 = _spmv(a, hl2, so=-dinv, mode="cheb", aux=tx1, si_next=dinv)
    tx4 = _spmv(a, hl3, so=-dinv, mode="cheb", aux=tx2)
    cat = jnp.concatenate([tx0, tx1, tx2, tx3, tx4], axis=1)
    wcat = cheb_W.reshape(5 * f, cheb_W.shape[2])
    h1 = _mm(cat, wcat, cheb_b, relu=True)

    z1 = jnp.zeros((gcn1_W.shape[1],), jnp.float32)
    z2 = jnp.zeros((gcn2_W.shape[1],), jnp.float32)
    vw1, vhl1 = _mm(h1, gcn1_W, z1, relu=False, si_next=dgc)
    h2 = _spmv(a, vhl1, so=dgc, mode="gcn", sl=dgc2, vown=vw1, b=gcn1_b)
    vw2, vhl2 = _mm(h2, gcn2_W, z2, relu=False, si_next=dgc)
    h3 = _spmv(a, vhl2, so=dgc, mode="gcn", sl=dgc2, vown=vw2, b=gcn2_b)

    return _pool(h3, batch_p, gate_w, gate_b, fc_W, fc_b, nb)


# speedup vs baseline: 1.0613x; 1.0059x over previous
import functools

import jax
import jax.numpy as jnp
from jax.experimental import pallas as pl

NPAD = 10240
BM = 1024
BK = 2048
SPLIT2 = False


def _split(s):
    hi = s.astype(jnp.bfloat16)
    lo = (s - hi.astype(jnp.float32)).astype(jnp.bfloat16)
    return hi, lo


def _emit_hl(s):
    if SPLIT2:
        hi, lo = _split(s)
        return jnp.concatenate([hi, lo], axis=1)
    return s.astype(jnp.bfloat16)


def _prep_body(v_ref, si_ref, hl_ref):
    hl_ref[...] = _emit_hl(v_ref[...] * si_ref[...])


def _prep(v, si):
    w = v.shape[1]
    return pl.pallas_call(
        _prep_body,
        grid=(NPAD // BM,),
        in_specs=[
            pl.BlockSpec((BM, w), lambda i: (i, 0)),
            pl.BlockSpec((BM, 1), lambda i: (i, 0)),
        ],
        out_specs=pl.BlockSpec((BM, (2 if SPLIT2 else 1) * w), lambda i: (i, 0)),
        out_shape=jax.ShapeDtypeStruct((NPAD, (2 if SPLIT2 else 1) * w), jnp.bfloat16),
    )(v, si.reshape(NPAD, 1))


def _spmv_body(*refs, mode, emit, convert):
    k = pl.program_id(1)
    nk = pl.num_programs(1)
    if mode == "gcn":
        (so_ref, a_ref, hl_ref, sl_ref, vown_ref, b_ref), rest = \
            refs[:6], refs[6:]
    elif mode == "cheb":
        (so_ref, a_ref, hl_ref, aux_ref), rest = refs[:4], refs[4:]
    else:
        (so_ref, a_ref, hl_ref), rest = refs[:3], refs[3:]
    p = 0
    if emit:
        sin_ref = rest[p]
        p += 1
    if convert:
        csi_ref = rest[p]
        p += 1
    outs = rest[p:]
    out_ref = outs[0]
    if emit:
        ohl_ref = outs[1]
    if convert:
        abf_ref = outs[-1]

    a = a_ref[...]
    if convert:
        a = a.astype(jnp.bfloat16)
        abf_ref[...] = a
        hl = _emit_hl(hl_ref[...] * csi_ref[...])
    else:
        hl = hl_ref[...]
    both = jnp.dot(a, hl, preferred_element_type=jnp.float32)
    if SPLIT2:
        w = hl_ref.shape[1] // 2
        part = both[:, :w] + both[:, w:]
    else:
        part = both

    @pl.when(k == 0)
    def _():
        out_ref[...] = part

    @pl.when(k > 0)
    def _():
        out_ref[...] += part

    @pl.when(k == nk - 1)
    def _():
        base = out_ref[...] * so_ref[...]
        if mode == "scale":
            res = base
        elif mode == "cheb":
            res = 2.0 * base - aux_ref[...]
        else:
            res = jax.nn.relu(base + sl_ref[...] * vown_ref[...] + b_ref[...])
        out_ref[...] = res
        if emit:
            ohl_ref[...] = _emit_hl(res * sin_ref[...])


def _spmv(a, hl, so, mode, aux=None, sl=None, vown=None, b=None,
          si_next=None, convert=False, convert_si=None):
    w = hl.shape[1] // (1 if convert else (2 if SPLIT2 else 1))
    grid = (NPAD // BM, NPAD // BK)
    emit = si_next is not None
    in_specs = [
        pl.BlockSpec((BM, 1), lambda i, k: (i, 0)),
        pl.BlockSpec((BM, BK), lambda i, k: (i, k)),
        pl.BlockSpec((BK, (2 if SPLIT2 else 1) * w), lambda i, k: (k, 0)),
    ]
    args = [so.reshape(NPAD, 1), a, hl]
    if mode == "cheb":
        in_specs.append(pl.BlockSpec((BM, w), lambda i, k: (i, 0)))
        args.append(aux)
    elif mode == "gcn":
        in_specs += [
            pl.BlockSpec((BM, 1), lambda i, k: (i, 0)),
            pl.BlockSpec((BM, w), lambda i, k: (i, 0)),
            pl.BlockSpec((1, w), lambda i, k: (0, 0)),
        ]
        args += [sl.reshape(NPAD, 1), vown, b.reshape(1, w)]
    if emit:
        in_specs.append(pl.BlockSpec((BM, 1), lambda i, k: (i, 0)))
        args.append(si_next.reshape(NPAD, 1))
        out_specs = [pl.BlockSpec((BM, w), lambda i, k: (i, 0)),
                     pl.BlockSpec((BM, (2 if SPLIT2 else 1) * w),
                                  lambda i, k: (i, 0))]
        out_shape = [jax.ShapeDtypeStruct((NPAD, w), jnp.float32),
                     jax.ShapeDtypeStruct((NPAD, (2 if SPLIT2 else 1) * w),
                                          jnp.bfloat16)]
    else:
        out_specs = [pl.BlockSpec((BM, w), lambda i, k: (i, 0))]
        out_shape = [jax.ShapeDtypeStruct((NPAD, w), jnp.float32)]
    if convert:
        in_specs[2] = pl.BlockSpec((BK, w), lambda i, k: (k, 0))
        in_specs.append(pl.BlockSpec((BK, 1), lambda i, k: (k, 0)))
        args.append(convert_si.reshape(NPAD, 1))
        out_specs = list(out_specs) if isinstance(out_specs, list) else [out_specs]
        out_specs.append(pl.BlockSpec((BM, BK), lambda i, k: (i, k)))
        out_shape = list(out_shape) if isinstance(out_shape, list) else [out_shape]
        out_shape.append(jax.ShapeDtypeStruct((NPAD, NPAD), jnp.bfloat16))

    res = pl.pallas_call(
        functools.partial(_spmv_body, mode=mode, emit=emit, convert=convert),
        grid=grid,
        in_specs=in_specs,
        out_specs=out_specs,
        out_shape=out_shape,
    )(*args)
    if not emit and not convert:
        return res[0]
    return res


def _cheb_mm_body(t0, t1, t2, t3, t4, wc_ref, b_ref, w1_ref, si_ref,
                  vw_ref, ohl_ref):
    f = t0.shape[1]
    acc = jnp.dot(t0[...], wc_ref[0 * f:1 * f, :],
                  preferred_element_type=jnp.float32)
    for j, t in enumerate((t1, t2, t3, t4), start=1):
        acc += jnp.dot(t[...], wc_ref[j * f:(j + 1) * f, :],
                       preferred_element_type=jnp.float32)
    h1 = jax.nn.relu(acc + b_ref[...])
    vw = jnp.dot(h1, w1_ref[...], preferred_element_type=jnp.float32)
    vw_ref[...] = vw
    ohl_ref[...] = _emit_hl(vw * si_ref[...])


def _cheb_mm(txs, wcat, b, w1, si_next):
    f = txs[0].shape[1]
    fo1 = wcat.shape[1]
    fo2 = w1.shape[1]
    in_specs = [pl.BlockSpec((BM, f), lambda i: (i, 0)) for _ in txs] + [
        pl.BlockSpec((5 * f, fo1), lambda i: (0, 0)),
        pl.BlockSpec((1, fo1), lambda i: (0, 0)),
        pl.BlockSpec((fo1, fo2), lambda i: (0, 0)),
        pl.BlockSpec((BM, 1), lambda i: (i, 0)),
    ]
    return pl.pallas_call(
        _cheb_mm_body,
        grid=(NPAD // BM,),
        in_specs=in_specs,
        out_specs=[pl.BlockSpec((BM, fo2), lambda i: (i, 0))] * 2,
        out_shape=[jax.ShapeDtypeStruct((NPAD, fo2), jnp.float32),
                   jax.ShapeDtypeStruct((NPAD, fo2), jnp.bfloat16)],
    )(*txs, wcat, b.reshape(1, fo1), w1, si_next.reshape(NPAD, 1))


def _mm_body(*refs, relu, emit):
    if emit:
        x_ref, w_ref, b_ref, si_ref, out_ref, ohl_ref = refs
    else:
        x_ref, w_ref, b_ref, out_ref = refs
    r = jnp.dot(x_ref[...], w_ref[...],
                preferred_element_type=jnp.float32) + b_ref[...]
    if relu:
        r = jax.nn.relu(r)
    out_ref[...] = r
    if emit:
        ohl_ref[...] = _emit_hl(r * si_ref[...])


def _mm(x, w, b, relu, si_next=None):
    fin, fout = w.shape
    emit = si_next is not None
    in_specs = [
        pl.BlockSpec((BM, fin), lambda i: (i, 0)),
        pl.BlockSpec((fin, fout), lambda i: (0, 0)),
        pl.BlockSpec((1, fout), lambda i: (0, 0)),
    ]
    args = [x, w, b.reshape(1, fout)]
    if emit:
        in_specs.append(pl.BlockSpec((BM, 1), lambda i: (i, 0)))
        args.append(si_next.reshape(NPAD, 1))
        out_specs = [pl.BlockSpec((BM, fout), lambda i: (i, 0)),
                     pl.BlockSpec((BM, (2 if SPLIT2 else 1) * fout),
                                  lambda i: (i, 0))]
        out_shape = [jax.ShapeDtypeStruct((NPAD, fout), jnp.float32),
                     jax.ShapeDtypeStruct((NPAD, (2 if SPLIT2 else 1) * fout),
                                          jnp.bfloat16)]
    else:
        out_specs = pl.BlockSpec((BM, fout), lambda i: (i, 0))
        out_shape = jax.ShapeDtypeStruct((NPAD, fout), jnp.float32)
    return pl.pallas_call(
        functools.partial(_mm_body, relu=relu, emit=emit),
        grid=(NPAD // BM,),
        in_specs=in_specs,
        out_specs=out_specs,
        out_shape=out_shape,
    )(*args)


def _pool_body(h_ref, batch_ref, gate_w_ref, gate_b_ref, fc_w_ref, fc_b_ref,
               out_ref, *, nb):
    h = h_ref[...]
    g = jnp.dot(h, gate_w_ref[...],
                preferred_element_type=jnp.float32) + gate_b_ref[...]
    seg = jax.lax.broadcasted_iota(jnp.int32, (h.shape[0], nb), 1)
    m = batch_ref[...] == seg
    neg = jnp.float32(-jnp.inf)
    gmax = jnp.max(jnp.where(m, g, neg), axis=0, keepdims=True)
    gmax = jnp.where(jnp.isfinite(gmax), gmax, 0.0)
    ge = jnp.where(m, jnp.exp(g - gmax), 0.0)
    gs = jnp.sum(ge, axis=0, keepdims=True)
    att = ge / jnp.maximum(gs, 1e-12)
    pooled = jax.lax.dot_general(att, h, (((0,), (0,)), ((), ())),
                                 preferred_element_type=jnp.float32)
    logits = jnp.dot(pooled, fc_w_ref[...],
                     preferred_element_type=jnp.float32) + fc_b_ref[...]
    mx = jnp.max(logits, axis=1, keepdims=True)
    lse = mx + jnp.log(jnp.sum(jnp.exp(logits - mx), axis=1, keepdims=True))
    out_ref[...] = logits - lse


def _pool(h, batch_padded, gate_w, gate_b, fc_w, fc_b, nb):
    c = fc_w.shape[1]
    return pl.pallas_call(
        functools.partial(_pool_body, nb=nb),
        out_shape=jax.ShapeDtypeStruct((nb, c), jnp.float32),
    )(h, batch_padded.reshape(NPAD, 1), gate_w, gate_b.reshape(1, 1),
      fc_w, fc_b.reshape(1, c))


def kernel(x, edge_index, batch, cheb_W, cheb_b, gcn1_W, gcn1_b,
           gcn2_W, gcn2_b, gate_w, gate_b, fc_W, fc_b):
    n, f = x.shape
    nb = 8
    row, col = edge_index[0], edge_index[1]

    base = NPAD * NPAD
    flat = col.astype(jnp.int32) * NPAD + row.astype(jnp.int32)
    idx = jnp.concatenate([flat, base + row, base + NPAD + col])
    buf = (jnp.zeros(((NPAD + 8) * NPAD,), jnp.float32)
           .at[idx].add(1.0).reshape(NPAD + 8, NPAD))
    deg_r = buf[NPAD]
    deg_c = buf[NPAD + 1]
    dinv = jnp.where(deg_r > 0, jax.lax.rsqrt(jnp.where(deg_r > 0, deg_r, 1.0)), 0.0)
    dgc = jax.lax.rsqrt(deg_c + 1.0)
    dgc2 = dgc * dgc

    xp = jnp.zeros((NPAD, f), jnp.float32).at[:n].set(x)
    batch_p = jnp.full((NPAD,), nb, jnp.int32).at[:n].set(batch)

    tx0 = xp
    tx1, hl1, a = _spmv(buf, xp, so=-dinv, mode="scale", si_next=dinv,
                        convert=True, convert_si=dinv)
    tx2, hl2 = _spmv(a, hl1, so=-dinv, mode="cheb", aux=tx0, si_next=dinv)
    tx3, hl3 = _spmv(a, hl2, so=-dinv, mode="cheb", aux=tx1, si_next=dinv)
    tx4 = _spmv(a, hl3, so=-dinv, mode="cheb", aux=tx2)
    wcat = cheb_W.reshape(5 * f, cheb_W.shape[2])
    vw1, vhl1 = _cheb_mm((tx0, tx1, tx2, tx3, tx4), wcat, cheb_b, gcn1_W, dgc)

    z2 = jnp.zeros((gcn2_W.shape[1],), jnp.float32)
    h2 = _spmv(a, vhl1, so=dgc, mode="gcn", sl=dgc2, vown=vw1, b=gcn1_b)
    vw2, vhl2 = _mm(h2, gcn2_W, z2, relu=False, si_next=dgc)
    h3 = _spmv(a, vhl2, so=dgc, mode="gcn", sl=dgc2, vown=vw2, b=gcn2_b)

    return _pool(h3, batch_p, gate_w, gate_b, fc_W, fc_b, nb)


# speedup vs baseline: 1.0754x; 1.0133x over previous
import functools

import jax
import jax.numpy as jnp
from jax.experimental import pallas as pl

NPAD = 10240
BM = 1024
BK = 2560
SPLIT2 = False


def _split(s):
    hi = s.astype(jnp.bfloat16)
    lo = (s - hi.astype(jnp.float32)).astype(jnp.bfloat16)
    return hi, lo


def _emit_hl(s):
    if SPLIT2:
        hi, lo = _split(s)
        return jnp.concatenate([hi, lo], axis=1)
    return s.astype(jnp.bfloat16)


def _prep_body(v_ref, si_ref, hl_ref):
    hl_ref[...] = _emit_hl(v_ref[...] * si_ref[...])


def _prep(v, si):
    w = v.shape[1]
    return pl.pallas_call(
        _prep_body,
        grid=(NPAD // BM,),
        in_specs=[
            pl.BlockSpec((BM, w), lambda i: (i, 0)),
            pl.BlockSpec((BM, 1), lambda i: (i, 0)),
        ],
        out_specs=pl.BlockSpec((BM, (2 if SPLIT2 else 1) * w), lambda i: (i, 0)),
        out_shape=jax.ShapeDtypeStruct((NPAD, (2 if SPLIT2 else 1) * w), jnp.bfloat16),
    )(v, si.reshape(NPAD, 1))


def _spmv_body(*refs, mode, emit, convert):
    k = pl.program_id(1)
    nk = pl.num_programs(1)
    if mode == "gcn":
        (so_ref, a_ref, hl_ref, sl_ref, vown_ref, b_ref), rest = \
            refs[:6], refs[6:]
    elif mode == "cheb":
        (so_ref, a_ref, hl_ref, aux_ref), rest = refs[:4], refs[4:]
    else:
        (so_ref, a_ref, hl_ref), rest = refs[:3], refs[3:]
    p = 0
    if emit:
        sin_ref = rest[p]
        p += 1
    if convert:
        csi_ref = rest[p]
        p += 1
    outs = rest[p:]
    out_ref = outs[0]
    if emit:
        ohl_ref = outs[1]
    if convert:
        abf_ref = outs[-1]

    a = a_ref[...]
    if convert:
        a = a.astype(jnp.bfloat16)
        abf_ref[...] = a
        hl = _emit_hl(hl_ref[...] * csi_ref[...])
    else:
        hl = hl_ref[...]
    both = jnp.dot(a, hl, preferred_element_type=jnp.float32)
    if SPLIT2:
        w = hl_ref.shape[1] // 2
        part = both[:, :w] + both[:, w:]
    else:
        part = both

    @pl.when(k == 0)
    def _():
        out_ref[...] = part

    @pl.when(k > 0)
    def _():
        out_ref[...] += part

    @pl.when(k == nk - 1)
    def _():
        base = out_ref[...] * so_ref[...]
        if mode == "scale":
            res = base
        elif mode == "cheb":
            res = 2.0 * base - aux_ref[...]
        else:
            res = jax.nn.relu(base + sl_ref[...] * vown_ref[...] + b_ref[...])
        out_ref[...] = res
        if emit:
            ohl_ref[...] = _emit_hl(res * sin_ref[...])


def _spmv(a, hl, so, mode, aux=None, sl=None, vown=None, b=None,
          si_next=None, convert=False, convert_si=None):
    w = hl.shape[1] // (1 if convert else (2 if SPLIT2 else 1))
    grid = (NPAD // BM, NPAD // BK)
    emit = si_next is not None
    in_specs = [
        pl.BlockSpec((BM, 1), lambda i, k: (i, 0)),
        pl.BlockSpec((BM, BK), lambda i, k: (i, k)),
        pl.BlockSpec((BK, (2 if SPLIT2 else 1) * w), lambda i, k: (k, 0)),
    ]
    args = [so.reshape(NPAD, 1), a, hl]
    if mode == "cheb":
        in_specs.append(pl.BlockSpec((BM, w), lambda i, k: (i, 0)))
        args.append(aux)
    elif mode == "gcn":
        in_specs += [
            pl.BlockSpec((BM, 1), lambda i, k: (i, 0)),
            pl.BlockSpec((BM, w), lambda i, k: (i, 0)),
            pl.BlockSpec((1, w), lambda i, k: (0, 0)),
        ]
        args += [sl.reshape(NPAD, 1), vown, b.reshape(1, w)]
    if emit:
        in_specs.append(pl.BlockSpec((BM, 1), lambda i, k: (i, 0)))
        args.append(si_next.reshape(NPAD, 1))
        out_specs = [pl.BlockSpec((BM, w), lambda i, k: (i, 0)),
                     pl.BlockSpec((BM, (2 if SPLIT2 else 1) * w),
                                  lambda i, k: (i, 0))]
        out_shape = [jax.ShapeDtypeStruct((NPAD, w), jnp.float32),
                     jax.ShapeDtypeStruct((NPAD, (2 if SPLIT2 else 1) * w),
                                          jnp.bfloat16)]
    else:
        out_specs = [pl.BlockSpec((BM, w), lambda i, k: (i, 0))]
        out_shape = [jax.ShapeDtypeStruct((NPAD, w), jnp.float32)]
    if convert:
        in_specs[2] = pl.BlockSpec((BK, w), lambda i, k: (k, 0))
        in_specs.append(pl.BlockSpec((BK, 1), lambda i, k: (k, 0)))
        args.append(convert_si.reshape(NPAD, 1))
        out_specs = list(out_specs) if isinstance(out_specs, list) else [out_specs]
        out_specs.append(pl.BlockSpec((BM, BK), lambda i, k: (i, k)))
        out_shape = list(out_shape) if isinstance(out_shape, list) else [out_shape]
        out_shape.append(jax.ShapeDtypeStruct((NPAD, NPAD), jnp.bfloat16))

    res = pl.pallas_call(
        functools.partial(_spmv_body, mode=mode, emit=emit, convert=convert),
        grid=grid,
        in_specs=in_specs,
        out_specs=out_specs,
        out_shape=out_shape,
    )(*args)
    if not emit and not convert:
        return res[0]
    return res


def _cheb_mm_body(t0, t1, t2, t3, t4, wc_ref, b_ref, w1_ref, si_ref,
                  vw_ref, ohl_ref):
    f = t0.shape[1]
    acc = jnp.dot(t0[...], wc_ref[0 * f:1 * f, :],
                  preferred_element_type=jnp.float32)
    for j, t in enumerate((t1, t2, t3, t4), start=1):
        acc += jnp.dot(t[...], wc_ref[j * f:(j + 1) * f, :],
                       preferred_element_type=jnp.float32)
    h1 = jax.nn.relu(acc + b_ref[...])
    vw = jnp.dot(h1, w1_ref[...], preferred_element_type=jnp.float32)
    vw_ref[...] = vw
    ohl_ref[...] = _emit_hl(vw * si_ref[...])


def _cheb_mm(txs, wcat, b, w1, si_next):
    f = txs[0].shape[1]
    fo1 = wcat.shape[1]
    fo2 = w1.shape[1]
    in_specs = [pl.BlockSpec((BM, f), lambda i: (i, 0)) for _ in txs] + [
        pl.BlockSpec((5 * f, fo1), lambda i: (0, 0)),
        pl.BlockSpec((1, fo1), lambda i: (0, 0)),
        pl.BlockSpec((fo1, fo2), lambda i: (0, 0)),
        pl.BlockSpec((BM, 1), lambda i: (i, 0)),
    ]
    return pl.pallas_call(
        _cheb_mm_body,
        grid=(NPAD // BM,),
        in_specs=in_specs,
        out_specs=[pl.BlockSpec((BM, fo2), lambda i: (i, 0))] * 2,
        out_shape=[jax.ShapeDtypeStruct((NPAD, fo2), jnp.float32),
                   jax.ShapeDtypeStruct((NPAD, fo2), jnp.bfloat16)],
    )(*txs, wcat, b.reshape(1, fo1), w1, si_next.reshape(NPAD, 1))


def _mm_body(*refs, relu, emit):
    if emit:
        x_ref, w_ref, b_ref, si_ref, out_ref, ohl_ref = refs
    else:
        x_ref, w_ref, b_ref, out_ref = refs
    r = jnp.dot(x_ref[...], w_ref[...],
                preferred_element_type=jnp.float32) + b_ref[...]
    if relu:
        r = jax.nn.relu(r)
    out_ref[...] = r
    if emit:
        ohl_ref[...] = _emit_hl(r * si_ref[...])


def _mm(x, w, b, relu, si_next=None):
    fin, fout = w.shape
    emit = si_next is not None
    in_specs = [
        pl.BlockSpec((BM, fin), lambda i: (i, 0)),
        pl.BlockSpec((fin, fout), lambda i: (0, 0)),
        pl.BlockSpec((1, fout), lambda i: (0, 0)),
    ]
    args = [x, w, b.reshape(1, fout)]
    if emit:
        in_specs.append(pl.BlockSpec((BM, 1), lambda i: (i, 0)))
        args.append(si_next.reshape(NPAD, 1))
        out_specs = [pl.BlockSpec((BM, fout), lambda i: (i, 0)),
                     pl.BlockSpec((BM, (2 if SPLIT2 else 1) * fout),
                                  lambda i: (i, 0))]
        out_shape = [jax.ShapeDtypeStruct((NPAD, fout), jnp.float32),
                     jax.ShapeDtypeStruct((NPAD, (2 if SPLIT2 else 1) * fout),
                                          jnp.bfloat16)]
    else:
        out_specs = pl.BlockSpec((BM, fout), lambda i: (i, 0))
        out_shape = jax.ShapeDtypeStruct((NPAD, fout), jnp.float32)
    return pl.pallas_call(
        functools.partial(_mm_body, relu=relu, emit=emit),
        grid=(NPAD // BM,),
        in_specs=in_specs,
        out_specs=out_specs,
        out_shape=out_shape,
    )(*args)


def _pool_body(h_ref, batch_ref, gate_w_ref, gate_b_ref, fc_w_ref, fc_b_ref,
               out_ref, *, nb):
    h = h_ref[...]
    g = jnp.dot(h, gate_w_ref[...],
                preferred_element_type=jnp.float32) + gate_b_ref[...]
    seg = jax.lax.broadcasted_iota(jnp.int32, (h.shape[0], nb), 1)
    m = batch_ref[...] == seg
    neg = jnp.float32(-jnp.inf)
    gmax = jnp.max(jnp.where(m, g, neg), axis=0, keepdims=True)
    gmax = jnp.where(jnp.isfinite(gmax), gmax, 0.0)
    ge = jnp.where(m, jnp.exp(g - gmax), 0.0)
    gs = jnp.sum(ge, axis=0, keepdims=True)
    att = ge / jnp.maximum(gs, 1e-12)
    pooled = jax.lax.dot_general(att, h, (((0,), (0,)), ((), ())),
                                 preferred_element_type=jnp.float32)
    logits = jnp.dot(pooled, fc_w_ref[...],
                     preferred_element_type=jnp.float32) + fc_b_ref[...]
    mx = jnp.max(logits, axis=1, keepdims=True)
    lse = mx + jnp.log(jnp.sum(jnp.exp(logits - mx), axis=1, keepdims=True))
    out_ref[...] = logits - lse


def _pool(h, batch_padded, gate_w, gate_b, fc_w, fc_b, nb):
    c = fc_w.shape[1]
    return pl.pallas_call(
        functools.partial(_pool_body, nb=nb),
        out_shape=jax.ShapeDtypeStruct((nb, c), jnp.float32),
    )(h, batch_padded.reshape(NPAD, 1), gate_w, gate_b.reshape(1, 1),
      fc_w, fc_b.reshape(1, c))


def kernel(x, edge_index, batch, cheb_W, cheb_b, gcn1_W, gcn1_b,
           gcn2_W, gcn2_b, gate_w, gate_b, fc_W, fc_b):
    n, f = x.shape
    nb = 8
    row, col = edge_index[0], edge_index[1]

    base = NPAD * NPAD
    flat = col.astype(jnp.int32) * NPAD + row.astype(jnp.int32)
    idx = jnp.concatenate([flat, base + row, base + NPAD + col])
    buf = (jnp.zeros(((NPAD + 8) * NPAD,), jnp.float32)
           .at[idx].add(1.0).reshape(NPAD + 8, NPAD))
    deg_r = buf[NPAD]
    deg_c = buf[NPAD + 1]
    dinv = jnp.where(deg_r > 0, jax.lax.rsqrt(jnp.where(deg_r > 0, deg_r, 1.0)), 0.0)
    dgc = jax.lax.rsqrt(deg_c + 1.0)
    dgc2 = dgc * dgc

    xp = jnp.zeros((NPAD, f), jnp.float32).at[:n].set(x)
    batch_p = jnp.full((NPAD,), nb, jnp.int32).at[:n].set(batch)

    tx0 = xp
    tx1, hl1, a = _spmv(buf, xp, so=-dinv, mode="scale", si_next=dinv,
                        convert=True, convert_si=dinv)
    tx2, hl2 = _spmv(a, hl1, so=-dinv, mode="cheb", aux=tx0, si_next=dinv)
    tx3, hl3 = _spmv(a, hl2, so=-dinv, mode="cheb", aux=tx1, si_next=dinv)
    tx4 = _spmv(a, hl3, so=-dinv, mode="cheb", aux=tx2)
    wcat = cheb_W.reshape(5 * f, cheb_W.shape[2])
    vw1, vhl1 = _cheb_mm((tx0, tx1, tx2, tx3, tx4), wcat, cheb_b, gcn1_W, dgc)

    z2 = jnp.zeros((gcn2_W.shape[1],), jnp.float32)
    h2 = _spmv(a, vhl1, so=dgc, mode="gcn", sl=dgc2, vown=vw1, b=gcn1_b)
    vw2, vhl2 = _mm(h2, gcn2_W, z2, relu=False, si_next=dgc)
    h3 = _spmv(a, vhl2, so=dgc, mode="gcn", sl=dgc2, vown=vw2, b=gcn2_b)

    return _pool(h3, batch_p, gate_w, gate_b, fc_W, fc_b, nb)
